# baseline (speedup 1.0000x reference)
import functools

import jax
import jax.numpy as jnp
import numpy as np
from jax import lax
from jax.experimental import pallas as pl
from jax.experimental.pallas import tpu as pltpu
from jax.experimental.pallas import tpu_sc as plsc

N = 10000
E = 300000
H = 32
LAYERS = 4

NC = 2
NS = 16
NW = NC * NS
CHUNK = 128
NCHUNK = 80
PER_TILE = CHUNK * NCHUNK
EPAD = PER_TILE * NW
NROW = N // NS


def _group(d, acc_rows=0):
    budget = 430 * 1024 - NCHUNK * CHUNK * 4 - acc_rows * d * 4 // NS
    cap = max(1, budget // ((CHUNK + 2) * d * 4 + CHUNK * 4))
    return max(g for g in (1, 2, 4, 5, 8, 10, 16) if g <= cap)


def _copy_idx(dst_ref, src_ref, row):
    for i in range(CHUNK // 16):
        dst_ref[pl.ds(i * 16, 16)] = src_ref[row, pl.ds(i * 16, 16)]

@functools.lru_cache(maxsize=1)
def _mesh():
    return plsc.VectorSubcoreMesh(core_axis_name="c", subcore_axis_name="s")


def _sc_gather(table, idx2d):
    n, d = table.shape
    g = _group(d)
    ngrp = NCHUNK // g

    @functools.partial(
        pl.kernel,
        out_type=jax.ShapeDtypeStruct((EPAD, d), jnp.float32),
        mesh=_mesh(),
        compiler_params=pltpu.CompilerParams(use_tc_tiling_on_sc=False),
        scratch_types=[
            pltpu.VMEM((NCHUNK, CHUNK), jnp.int32),
            pltpu.VMEM((g * CHUNK, d), jnp.float32),
            pltpu.SemaphoreType.DMA,
        ] + [pltpu.VMEM((CHUNK,), jnp.int32)] * g,
    )
    def k(table_hbm, idx_hbm, out_hbm, idx_v, rows_v, sem, *idx_c):
        wid = lax.axis_index("s") * NC + lax.axis_index("c")
        pltpu.sync_copy(idx_hbm.at[pl.ds(wid * NCHUNK, NCHUNK)], idx_v)

        def body(j, carry):
            for b in range(g):
                _copy_idx(idx_c[b], idx_v, j * g + b)
            descs = [
                pltpu.async_copy(table_hbm.at[idx_c[b]],
                                 rows_v.at[pl.ds(b * CHUNK, CHUNK)], sem)
                for b in range(g)
            ]
            for dsc in descs:
                dsc.wait()
            base = wid * PER_TILE + j * (g * CHUNK)
            pltpu.sync_copy(rows_v, out_hbm.at[pl.ds(base, g * CHUNK)])
            return carry

        lax.fori_loop(0, ngrp, body, 0)

    return k(table, idx2d)


def _sc_scatter_add(rows, idx2d):
    d = rows.shape[1]
    g = _group(d, acc_rows=N)
    ngrp = NCHUNK // g
    zeros = jnp.zeros((N, d), jnp.float32)

    @functools.partial(
        pl.kernel,
        out_type=jax.ShapeDtypeStruct((NC, N, d), jnp.float32),
        mesh=_mesh(),
        compiler_params=pltpu.CompilerParams(use_tc_tiling_on_sc=False),
        scratch_types=[
            pltpu.VMEM((NCHUNK, CHUNK), jnp.int32),
            pltpu.VMEM((g * CHUNK, d), jnp.float32),
            pltpu.VMEM_SHARED((N, d), jnp.float32),
            pltpu.SemaphoreType.DMA,
            pltpu.SemaphoreType.DMA,
        ] + [pltpu.VMEM((CHUNK,), jnp.int32)] * g,
    )
    def k(rows_hbm, idx_hbm, zero_hbm, out_hbm, idx_v, rows_v, acc_sh,
          sem, sem2, *idx_c):
        cid = lax.axis_index("c")
        sid = lax.axis_index("s")
        wid = sid * NC + cid
        pltpu.sync_copy(idx_hbm.at[pl.ds(wid * NCHUNK, NCHUNK)], idx_v)
        pltpu.sync_copy(zero_hbm.at[pl.ds(sid * NROW, NROW)],
                        acc_sh.at[pl.ds(sid * NROW, NROW)])
        plsc.subcore_barrier()

        def body(j, carry):
            base = wid * PER_TILE + j * (g * CHUNK)
            ld = pltpu.async_copy(rows_hbm.at[pl.ds(base, g * CHUNK)],
                                  rows_v, sem)
            for b in range(g):
                _copy_idx(idx_c[b], idx_v, j * g + b)
            ld.wait()
            descs = [
                pltpu.async_copy(rows_v.at[pl.ds(b * CHUNK, CHUNK)],
                                 acc_sh.at[idx_c[b]], sem2, add=True)
                for b in range(g)
            ]
            for dsc in descs:
                dsc.wait()
            return carry

        lax.fori_loop(0, ngrp, body, 0)
        plsc.subcore_barrier()
        pltpu.sync_copy(acc_sh.at[pl.ds(sid * NROW, NROW)],
                        out_hbm.at[cid, pl.ds(sid * NROW, NROW)])

    return k(rows, idx2d, zeros)


def _segsum(rows, idx):
    parts = _sc_scatter_add(rows, idx)
    return parts[0] + parts[1]


def _normalize(v, axis=-1, eps=1e-8):
    return v / (jnp.linalg.norm(v, axis=axis, keepdims=True) + eps)


def _featurize(bb):
    X_ca = bb[:, 1, :]
    X = bb[:, :3, :].reshape(-1, 3)
    dX = X[1:] - X[:-1]
    U = _normalize(dX)
    u2 = U[:-2]
    u1 = U[1:-1]
    u0 = U[2:]
    n2 = _normalize(jnp.cross(u2, u1))
    n1 = _normalize(jnp.cross(u1, u0))
    cosD = jnp.clip(jnp.sum(n2 * n1, -1), -1 + 1e-7, 1 - 1e-7)
    D = jnp.sign(jnp.sum(u2 * n1, -1)) * jnp.arccos(cosD)
    D = jnp.pad(D, (1, 2)).reshape(-1, 3)
    dih = jnp.concatenate([jnp.cos(D), jnp.sin(D)], axis=-1)

    fwd = jnp.pad(_normalize(X_ca[1:] - X_ca[:-1]), ((0, 1), (0, 0)))
    bwd = jnp.pad(_normalize(X_ca[:-1] - X_ca[1:]), ((1, 0), (0, 0)))
    ori = jnp.stack([fwd, bwd], axis=-2)

    Nn = bb[:, 0, :]
    Ca = bb[:, 1, :]
    C = bb[:, 2, :]
    b = Ca - Nn
    c = C - Ca
    a = jnp.cross(b, c)
    vcb = (-0.58273431 * a + 0.56802827 * b - 0.54067466 * c + Ca) - X_ca

    bb_rel = bb - X_ca[:, None, :]
    l1 = jnp.nan_to_num(
        jnp.concatenate([bb_rel, ori, vcb[:, None, :]], axis=-2))

    emb = jnp.zeros((N, 4, 26), jnp.float32)
    emb = emb.at[:, 0, :6].set(dih)
    emb = emb.at[:, 1:4, :7].set(jnp.transpose(l1, (0, 2, 1)))
    return emb, X_ca


def _rbf(d):
    mu = jnp.linspace(0.0, 20.0, 16)
    sigma = 20.0 / 16
    return jnp.exp(-(((d[:, None] - mu) / sigma) ** 2))


def _pad_cols(x, d):
    return jnp.pad(x, ((0, 0), (0, d - x.shape[1])))


def kernel(bb, seq, latent, edge_index, params):
    src = edge_index[0].astype(jnp.int32)
    dst = edge_index[1].astype(jnp.int32)
    pad = EPAD - E
    src_p = jnp.pad(src, (0, pad))
    dst_p = jnp.pad(dst, (0, pad))
    src2 = src_p.reshape(-1, CHUNK)
    dst2 = dst_p.reshape(-1, CHUNK)
    emask = jnp.pad(jnp.ones((E,), jnp.float32), (0, pad))[:, None]

    emb, X_ca = _featurize(bb)

    xca16 = _pad_cols(X_ca, 16)
    xs = _sc_gather(xca16, src2)[:, :3]
    xd = _sc_gather(xca16, dst2)[:, :3]
    dist = jnp.linalg.norm(xd - xs, axis=-1)
    dpos = (src_p - dst_p).astype(jnp.float32)
    freq = jnp.exp(jnp.arange(0, 16, 2, dtype=jnp.float32) * (-np.log(10000.0) / 16))
    ang = dpos[:, None] * freq[None, :]
    e = jnp.concatenate([_rbf(dist), jnp.cos(ang), jnp.sin(ang)], axis=-1)

    emb28 = jnp.pad(emb, ((0, 0), (0, 0), (0, 2)))
    embflat = emb28.reshape(N, 112)
    gate = jax.nn.sigmoid(e @ params['Wg_bb'] + params['bg_bb'])
    rows = _sc_gather(embflat, src2)
    w = rows * jnp.tile(_pad_cols(gate, 28), (1, 4)) * emask
    agg = _segsum(w, dst2).reshape(N, 4, 28)[:, :, :26]
    bb_emb = agg @ params['Wo_bb'] + emb @ params['Ws_bb']

    x = jnp.concatenate([bb_emb, latent], axis=-1)

    for i in range(LAYERS):
        x_inv = x[:, 0, :]
        q = x_inv @ params['Wq'][i]
        kk = x_inv @ params['Wk'][i]
        qg = _sc_gather(q, dst2)
        kg = _sc_gather(kk, src2)
        score = jnp.sum(qg * kg, -1) / jnp.sqrt(16.0) + e @ params['We_att'][i]
        m = jnp.max(jnp.where(emask[:, 0] > 0, score, -1e30))
        ex = jnp.exp(score - m) * emask[:, 0]

        x_flat = x.reshape(N, 256)
        x_aug = jnp.concatenate(
            [x_flat, jnp.ones((N, 1), jnp.float32), jnp.zeros((N, 15), jnp.float32)],
            axis=-1)
        rows = _sc_gather(x_aug, src2) * ex[:, None]
        agg = jnp.concatenate(
            [_segsum(rows[:, :96], dst2), _segsum(rows[:, 96:192], dst2),
             _segsum(rows[:, 192:], dst2)], axis=-1)
        num = agg[:, :256].reshape(N, 4, 64)
        den = agg[:, 256:257]
        attn = (num / (den[:, :, None] + 1e-9)) @ params['Wv'][i]
        x = x + attn
        x = x + jax.nn.relu(x @ params['W1'][i]) @ params['W2'][i]

        x_inv = x[:, 0, :]
        ps = x_inv @ params['We1'][i][:64]
        pd = x_inv @ params['We1'][i][64:128]
        psg = _sc_gather(ps, src2)
        pdg = _sc_gather(pd, dst2)
        e = e + jax.nn.relu(psg + pdg + e @ params['We1'][i][128:] + params['be1'][i])

    x_flat = x.reshape(N, 256)
    gate = jax.nn.sigmoid(e @ params['Wg_p'] + params['bg_p'])
    rows = _sc_gather(x_flat, src2) * jnp.tile(gate, (1, 4)) * emask
    agg = jnp.concatenate(
        [_segsum(rows[:, :128], dst2), _segsum(rows[:, 128:], dst2)], axis=-1)
    x2 = agg.reshape(N, 4, 64) @ params['Wo_p'] + x @ params['Ws_p']

    x2_flat = x2.reshape(N, 128)
    gate = jax.nn.sigmoid(e @ params['Wg_a'] + params['bg_a'])
    rows = _sc_gather(x2_flat, src2) * jnp.tile(gate, (1, 4)) * emask
    agg = _segsum(rows, dst2).reshape(N, 4, 32)
    a = agg @ params['Wo_a'] + x2 @ params['Ws_a']

    inv = a[:, 0, :]
    mu = jnp.mean(inv, -1, keepdims=True)
    var = jnp.var(inv, -1, keepdims=True)
    ln = (inv - mu) / jnp.sqrt(var + 1e-5) * params['ln_g'] + params['ln_b']
    seq_logits = jax.nn.log_softmax(ln @ params['W_seq'] + params['b_seq'], axis=-1)
    decoded_latent = jnp.transpose(a[:, 1:4, :], (0, 2, 1))
    return decoded_latent, seq_logits


# speedup vs baseline: 1.0013x; 1.0013x over previous
import functools

import jax
import jax.numpy as jnp
import numpy as np
from jax import lax
from jax.experimental import pallas as pl
from jax.experimental.pallas import tpu as pltpu
from jax.experimental.pallas import tpu_sc as plsc

N = 10000
E = 300000
H = 32
LAYERS = 4

NC = 2
NS = 16
NW = NC * NS
CHUNK = 128
NCHUNK = 80
PER_TILE = CHUNK * NCHUNK
EPAD = PER_TILE * NW
NROW = N // NS


@functools.lru_cache(maxsize=1)
def _mesh():
    return plsc.VectorSubcoreMesh(core_axis_name="c", subcore_axis_name="s")


def _sc_gather(table, idx):
    n, d = table.shape

    @functools.partial(
        pl.kernel,
        out_type=jax.ShapeDtypeStruct((EPAD, d), jnp.float32),
        mesh=_mesh(),
        compiler_params=pltpu.CompilerParams(use_tc_tiling_on_sc=False),
        scratch_types=[
            pltpu.VMEM((CHUNK,), jnp.int32),
            pltpu.VMEM((CHUNK, d), jnp.float32),
            pltpu.SemaphoreType.DMA,
        ],
    )
    def k(table_hbm, idx_hbm, out_hbm, idx_v, rows_v, sem):
        wid = lax.axis_index("s") * NC + lax.axis_index("c")

        def body(j, carry):
            base = wid * PER_TILE + j * CHUNK
            pltpu.sync_copy(idx_hbm.at[pl.ds(base, CHUNK)], idx_v)
            pltpu.async_copy(table_hbm.at[idx_v], rows_v, sem).wait()
            pltpu.sync_copy(rows_v, out_hbm.at[pl.ds(base, CHUNK)])
            return carry

        lax.fori_loop(0, NCHUNK, body, 0)

    return k(table, idx)


def _sc_scatter_add(rows, idx):
    d = rows.shape[1]
    zeros = jnp.zeros((N, d), jnp.float32)

    @functools.partial(
        pl.kernel,
        out_type=jax.ShapeDtypeStruct((NC, N, d), jnp.float32),
        mesh=_mesh(),
        compiler_params=pltpu.CompilerParams(use_tc_tiling_on_sc=False),
        scratch_types=[
            pltpu.VMEM((CHUNK,), jnp.int32),
            pltpu.VMEM((CHUNK, d), jnp.float32),
            pltpu.VMEM_SHARED((N, d), jnp.float32),
            pltpu.SemaphoreType.DMA,
        ],
    )
    def k(rows_hbm, idx_hbm, zero_hbm, out_hbm, idx_v, rows_v, acc_sh, sem):
        cid = lax.axis_index("c")
        sid = lax.axis_index("s")
        wid = sid * NC + cid
        pltpu.sync_copy(zero_hbm.at[pl.ds(sid * NROW, NROW)],
                        acc_sh.at[pl.ds(sid * NROW, NROW)])
        plsc.subcore_barrier()

        def body(j, carry):
            base = wid * PER_TILE + j * CHUNK
            pltpu.sync_copy(idx_hbm.at[pl.ds(base, CHUNK)], idx_v)
            pltpu.sync_copy(rows_hbm.at[pl.ds(base, CHUNK)], rows_v)
            pltpu.sync_copy(rows_v, acc_sh.at[idx_v], add=True)
            return carry

        lax.fori_loop(0, NCHUNK, body, 0)
        plsc.subcore_barrier()
        pltpu.sync_copy(acc_sh.at[pl.ds(sid * NROW, NROW)],
                        out_hbm.at[cid, pl.ds(sid * NROW, NROW)])

    return k(rows, idx, zeros)


def _segsum(rows, idx):
    parts = _sc_scatter_add(rows, idx)
    return parts[0] + parts[1]


def _normalize(v, axis=-1, eps=1e-8):
    return v / (jnp.linalg.norm(v, axis=axis, keepdims=True) + eps)


def _featurize(bb):
    X_ca = bb[:, 1, :]
    X = bb[:, :3, :].reshape(-1, 3)
    dX = X[1:] - X[:-1]
    U = _normalize(dX)
    u2 = U[:-2]
    u1 = U[1:-1]
    u0 = U[2:]
    n2 = _normalize(jnp.cross(u2, u1))
    n1 = _normalize(jnp.cross(u1, u0))
    cosD = jnp.clip(jnp.sum(n2 * n1, -1), -1 + 1e-7, 1 - 1e-7)
    D = jnp.sign(jnp.sum(u2 * n1, -1)) * jnp.arccos(cosD)
    D = jnp.pad(D, (1, 2)).reshape(-1, 3)
    dih = jnp.concatenate([jnp.cos(D), jnp.sin(D)], axis=-1)

    fwd = jnp.pad(_normalize(X_ca[1:] - X_ca[:-1]), ((0, 1), (0, 0)))
    bwd = jnp.pad(_normalize(X_ca[:-1] - X_ca[1:]), ((1, 0), (0, 0)))
    ori = jnp.stack([fwd, bwd], axis=-2)

    Nn = bb[:, 0, :]
    Ca = bb[:, 1, :]
    C = bb[:, 2, :]
    b = Ca - Nn
    c = C - Ca
    a = jnp.cross(b, c)
    vcb = (-0.58273431 * a + 0.56802827 * b - 0.54067466 * c + Ca) - X_ca

    bb_rel = bb - X_ca[:, None, :]
    l1 = jnp.nan_to_num(
        jnp.concatenate([bb_rel, ori, vcb[:, None, :]], axis=-2))

    emb = jnp.zeros((N, 4, 26), jnp.float32)
    emb = emb.at[:, 0, :6].set(dih)
    emb = emb.at[:, 1:4, :7].set(jnp.transpose(l1, (0, 2, 1)))
    return emb, X_ca


def _rbf(d):
    mu = jnp.linspace(0.0, 20.0, 16)
    sigma = 20.0 / 16
    return jnp.exp(-(((d[:, None] - mu) / sigma) ** 2))


def _pad_cols(x, d):
    return jnp.pad(x, ((0, 0), (0, d - x.shape[1])))


def kernel(bb, seq, latent, edge_index, params):
    src = edge_index[0].astype(jnp.int32)
    dst = edge_index[1].astype(jnp.int32)
    pad = EPAD - E
    src_p = jnp.pad(src, (0, pad))
    dst_p = jnp.pad(dst, (0, pad))
    emask = jnp.pad(jnp.ones((E,), jnp.float32), (0, pad))[:, None]

    emb, X_ca = _featurize(bb)

    xca16 = _pad_cols(X_ca, 16)
    xs = _sc_gather(xca16, src_p)[:, :3]
    xd = _sc_gather(xca16, dst_p)[:, :3]
    dist = jnp.linalg.norm(xd - xs, axis=-1)
    dpos = (src_p - dst_p).astype(jnp.float32)
    freq = jnp.exp(jnp.arange(0, 16, 2, dtype=jnp.float32) * (-np.log(10000.0) / 16))
    ang = dpos[:, None] * freq[None, :]
    e = jnp.concatenate([_rbf(dist), jnp.cos(ang), jnp.sin(ang)], axis=-1)

    emb28 = jnp.pad(emb, ((0, 0), (0, 0), (0, 2)))
    embflat = emb28.reshape(N, 112)
    gate = jax.nn.sigmoid(e @ params['Wg_bb'] + params['bg_bb'])
    rows = _sc_gather(embflat, src_p)
    w = rows * jnp.tile(_pad_cols(gate, 28), (1, 4)) * emask
    agg = _segsum(w, dst_p).reshape(N, 4, 28)[:, :, :26]
    bb_emb = agg @ params['Wo_bb'] + emb @ params['Ws_bb']

    x = jnp.concatenate([bb_emb, latent], axis=-1)

    for i in range(LAYERS):
        x_inv = x[:, 0, :]
        q = x_inv @ params['Wq'][i]
        kk = x_inv @ params['Wk'][i]
        qg = _sc_gather(q, dst_p)
        kg = _sc_gather(kk, src_p)
        score = jnp.sum(qg * kg, -1) / jnp.sqrt(16.0) + e @ params['We_att'][i]
        m = jnp.max(jnp.where(emask[:, 0] > 0, score, -1e30))
        ex = jnp.exp(score - m) * emask[:, 0]

        x_flat = x.reshape(N, 256)
        x_aug = jnp.concatenate(
            [x_flat, jnp.ones((N, 1), jnp.float32), jnp.zeros((N, 15), jnp.float32)],
            axis=-1)
        rows = _sc_gather(x_aug, src_p) * ex[:, None]
        agg = jnp.concatenate(
            [_segsum(rows[:, :144], dst_p), _segsum(rows[:, 144:], dst_p)], axis=-1)
        num = agg[:, :256].reshape(N, 4, 64)
        den = agg[:, 256:257]
        attn = (num / (den[:, :, None] + 1e-9)) @ params['Wv'][i]
        x = x + attn
        x = x + jax.nn.relu(x @ params['W1'][i]) @ params['W2'][i]

        x_inv = x[:, 0, :]
        ps = x_inv @ params['We1'][i][:64]
        pd = x_inv @ params['We1'][i][64:128]
        psg = _sc_gather(ps, src_p)
        pdg = _sc_gather(pd, dst_p)
        e = e + jax.nn.relu(psg + pdg + e @ params['We1'][i][128:] + params['be1'][i])

    x_flat = x.reshape(N, 256)
    gate = jax.nn.sigmoid(e @ params['Wg_p'] + params['bg_p'])
    rows = _sc_gather(x_flat, src_p) * jnp.tile(gate, (1, 4)) * emask
    agg = jnp.concatenate(
        [_segsum(rows[:, :128], dst_p), _segsum(rows[:, 128:], dst_p)], axis=-1)
    x2 = agg.reshape(N, 4, 64) @ params['Wo_p'] + x @ params['Ws_p']

    x2_flat = x2.reshape(N, 128)
    gate = jax.nn.sigmoid(e @ params['Wg_a'] + params['bg_a'])
    rows = _sc_gather(x2_flat, src_p) * jnp.tile(gate, (1, 4)) * emask
    agg = _segsum(rows, dst_p).reshape(N, 4, 32)
    a = agg @ params['Wo_a'] + x2 @ params['Ws_a']

    inv = a[:, 0, :]
    mu = jnp.mean(inv, -1, keepdims=True)
    var = jnp.var(inv, -1, keepdims=True)
    ln = (inv - mu) / jnp.sqrt(var + 1e-5) * params['ln_g'] + params['ln_b']
    seq_logits = jax.nn.log_softmax(ln @ params['W_seq'] + params['b_seq'], axis=-1)
    decoded_latent = jnp.transpose(a[:, 1:4, :], (0, 2, 1))
    return decoded_latent, seq_logits


# speedup vs baseline: 1.4601x; 1.4581x over previous
import functools

import jax
import jax.numpy as jnp
import numpy as np
from jax import lax
from jax.experimental import pallas as pl
from jax.experimental.pallas import tpu as pltpu
from jax.experimental.pallas import tpu_sc as plsc

N = 10000
E = 300000
H = 32
LAYERS = 4

NC = 2
NS = 16
NW = NC * NS
CHUNK = 128
NCHUNK = 75
PER_TILE = CHUNK * NCHUNK
EPAD = PER_TILE * NW
NROW = N // NS


@functools.lru_cache(maxsize=1)
def _mesh():
    return plsc.VectorSubcoreMesh(core_axis_name="c", subcore_axis_name="s")


def _sc_gather(table, idx):
    n, d = table.shape

    @functools.partial(
        pl.kernel,
        out_type=jax.ShapeDtypeStruct((EPAD, d), jnp.float32),
        mesh=_mesh(),
        compiler_params=pltpu.CompilerParams(use_tc_tiling_on_sc=False),
        scratch_types=[
            pltpu.VMEM((CHUNK,), jnp.int32),
            pltpu.VMEM((CHUNK, d), jnp.float32),
            pltpu.SemaphoreType.DMA,
        ],
    )
    def k(table_hbm, idx_hbm, out_hbm, idx_v, rows_v, sem):
        wid = lax.axis_index("s") * NC + lax.axis_index("c")

        def body(j, carry):
            base = wid * PER_TILE + j * CHUNK
            pltpu.sync_copy(idx_hbm.at[pl.ds(base, CHUNK)], idx_v)
            pltpu.async_copy(table_hbm.at[idx_v], rows_v, sem).wait()
            pltpu.sync_copy(rows_v, out_hbm.at[pl.ds(base, CHUNK)])
            return carry

        lax.fori_loop(0, NCHUNK, body, 0)

    return k(table, idx)


def _sc_scatter_add(rows, idx):
    d = rows.shape[1]
    zeros = jnp.zeros((N, d), jnp.float32)

    @functools.partial(
        pl.kernel,
        out_type=jax.ShapeDtypeStruct((NC, N, d), jnp.float32),
        mesh=_mesh(),
        compiler_params=pltpu.CompilerParams(use_tc_tiling_on_sc=False),
        scratch_types=[
            pltpu.VMEM((CHUNK,), jnp.int32),
            pltpu.VMEM((CHUNK, d), jnp.float32),
            pltpu.VMEM_SHARED((N, d), jnp.float32),
            pltpu.SemaphoreType.DMA,
        ],
    )
    def k(rows_hbm, idx_hbm, zero_hbm, out_hbm, idx_v, rows_v, acc_sh, sem):
        cid = lax.axis_index("c")
        sid = lax.axis_index("s")
        wid = sid * NC + cid
        pltpu.sync_copy(zero_hbm.at[pl.ds(sid * NROW, NROW)],
                        acc_sh.at[pl.ds(sid * NROW, NROW)])
        plsc.subcore_barrier()

        def body(j, carry):
            base = wid * PER_TILE + j * CHUNK
            pltpu.sync_copy(idx_hbm.at[pl.ds(base, CHUNK)], idx_v)
            pltpu.sync_copy(rows_hbm.at[pl.ds(base, CHUNK)], rows_v)
            pltpu.sync_copy(rows_v, acc_sh.at[idx_v], add=True)
            return carry

        lax.fori_loop(0, NCHUNK, body, 0)
        plsc.subcore_barrier()
        pltpu.sync_copy(acc_sh.at[pl.ds(sid * NROW, NROW)],
                        out_hbm.at[cid, pl.ds(sid * NROW, NROW)])

    return k(rows, idx, zeros)


def _segsum(rows, idx):
    parts = _sc_scatter_add(rows, idx)
    return parts[0] + parts[1]


def _normalize(v, axis=-1, eps=1e-8):
    return v / (jnp.linalg.norm(v, axis=axis, keepdims=True) + eps)


def _featurize(bb):
    X_ca = bb[:, 1, :]
    X = bb[:, :3, :].reshape(-1, 3)
    dX = X[1:] - X[:-1]
    U = _normalize(dX)
    u2 = U[:-2]
    u1 = U[1:-1]
    u0 = U[2:]
    n2 = _normalize(jnp.cross(u2, u1))
    n1 = _normalize(jnp.cross(u1, u0))
    cosD = jnp.clip(jnp.sum(n2 * n1, -1), -1 + 1e-7, 1 - 1e-7)
    D = jnp.sign(jnp.sum(u2 * n1, -1)) * jnp.arccos(cosD)
    D = jnp.pad(D, (1, 2)).reshape(-1, 3)
    dih = jnp.concatenate([jnp.cos(D), jnp.sin(D)], axis=-1)

    fwd = jnp.pad(_normalize(X_ca[1:] - X_ca[:-1]), ((0, 1), (0, 0)))
    bwd = jnp.pad(_normalize(X_ca[:-1] - X_ca[1:]), ((1, 0), (0, 0)))
    ori = jnp.stack([fwd, bwd], axis=-2)

    Nn = bb[:, 0, :]
    Ca = bb[:, 1, :]
    C = bb[:, 2, :]
    b = Ca - Nn
    c = C - Ca
    a = jnp.cross(b, c)
    vcb = (-0.58273431 * a + 0.56802827 * b - 0.54067466 * c + Ca) - X_ca

    bb_rel = bb - X_ca[:, None, :]
    l1 = jnp.nan_to_num(
        jnp.concatenate([bb_rel, ori, vcb[:, None, :]], axis=-2))

    emb = jnp.zeros((N, 4, 26), jnp.float32)
    emb = emb.at[:, 0, :6].set(dih)
    emb = emb.at[:, 1:4, :7].set(jnp.transpose(l1, (0, 2, 1)))
    return emb, X_ca


def _rbf(d):
    mu = jnp.linspace(0.0, 20.0, 16)
    sigma = 20.0 / 16
    return jnp.exp(-(((d[:, None] - mu) / sigma) ** 2))


def _pad_cols(x, d):
    return jnp.pad(x, ((0, 0), (0, d - x.shape[1])))


def kernel(bb, seq, latent, edge_index, params):
    src = edge_index[0].astype(jnp.int32)
    dst = edge_index[1].astype(jnp.int32)
    pad = EPAD - E
    src_p = jnp.pad(src, (0, pad))
    dst_p = jnp.pad(dst, (0, pad))
    emask = jnp.pad(jnp.ones((E,), jnp.float32), (0, pad))[:, None]

    emb, X_ca = _featurize(bb)

    xca16 = _pad_cols(X_ca, 16)
    xs = _sc_gather(xca16, src_p)[:, :3]
    xd = _sc_gather(xca16, dst_p)[:, :3]
    dist = jnp.linalg.norm(xd - xs, axis=-1)
    dpos = (src_p - dst_p).astype(jnp.float32)
    freq = jnp.exp(jnp.arange(0, 16, 2, dtype=jnp.float32) * (-np.log(10000.0) / 16))
    ang = dpos[:, None] * freq[None, :]
    e = jnp.concatenate([_rbf(dist), jnp.cos(ang), jnp.sin(ang)], axis=-1)

    emb28 = jnp.pad(emb, ((0, 0), (0, 0), (0, 2)))
    embflat = emb28.reshape(N, 112)
    gate = jax.nn.sigmoid(e @ params['Wg_bb'] + params['bg_bb'])
    rows = _sc_gather(embflat, src_p)
    w = rows * jnp.tile(_pad_cols(gate, 28), (1, 4)) * emask
    agg = _segsum(w, dst_p).reshape(N, 4, 28)[:, :, :26]
    bb_emb = agg @ params['Wo_bb'] + emb @ params['Ws_bb']

    x = jnp.concatenate([bb_emb, latent], axis=-1)

    for i in range(LAYERS):
        x_inv = x[:, 0, :]
        q = x_inv @ params['Wq'][i]
        kk = x_inv @ params['Wk'][i]
        qg = _sc_gather(q, dst_p)
        kg = _sc_gather(kk, src_p)
        score = jnp.sum(qg * kg, -1) / jnp.sqrt(16.0) + e @ params['We_att'][i]
        m = jnp.max(jnp.where(emask[:, 0] > 0, score, -1e30))
        ex = jnp.exp(score - m) * emask[:, 0]

        x_flat = x.reshape(N, 256)
        x_aug = jnp.concatenate(
            [x_flat, jnp.ones((N, 1), jnp.float32), jnp.zeros((N, 15), jnp.float32)],
            axis=-1)
        rows = _sc_gather(x_aug, src_p) * ex[:, None]
        agg = jnp.concatenate(
            [_segsum(rows[:, :144], dst_p), _segsum(rows[:, 144:], dst_p)], axis=-1)
        num = agg[:, :256].reshape(N, 4, 64)
        den = agg[:, 256:257]
        attn = (num / (den[:, :, None] + 1e-9)) @ params['Wv'][i]
        x = x + attn
        x = x + jax.nn.relu(x @ params['W1'][i]) @ params['W2'][i]

        x_inv = x[:, 0, :]
        ps = x_inv @ params['We1'][i][:64]
        pd = x_inv @ params['We1'][i][64:128]
        psg = _sc_gather(ps, src_p)
        pdg = _sc_gather(pd, dst_p)
        e = e + jax.nn.relu(psg + pdg + e @ params['We1'][i][128:] + params['be1'][i])

    x_flat = x.reshape(N, 256)
    gate = jax.nn.sigmoid(e @ params['Wg_p'] + params['bg_p'])
    rows = _sc_gather(x_flat, src_p) * jnp.tile(gate, (1, 4)) * emask
    agg = jnp.concatenate(
        [_segsum(rows[:, :128], dst_p), _segsum(rows[:, 128:], dst_p)], axis=-1)
    x2 = agg.reshape(N, 4, 64) @ params['Wo_p'] + x @ params['Ws_p']

    x2_flat = x2.reshape(N, 128)
    gate = jax.nn.sigmoid(e @ params['Wg_a'] + params['bg_a'])
    rows = _sc_gather(x2_flat, src_p) * jnp.tile(gate, (1, 4)) * emask
    agg = _segsum(rows, dst_p).reshape(N, 4, 32)
    a = agg @ params['Wo_a'] + x2 @ params['Ws_a']

    inv = a[:, 0, :]
    mu = jnp.mean(inv, -1, keepdims=True)
    var = jnp.var(inv, -1, keepdims=True)
    ln = (inv - mu) / jnp.sqrt(var + 1e-5) * params['ln_g'] + params['ln_b']
    seq_logits = jax.nn.log_softmax(ln @ params['W_seq'] + params['b_seq'], axis=-1)
    decoded_latent = jnp.transpose(a[:, 1:4, :], (0, 2, 1))
    return decoded_latent, seq_logits


# speedup vs baseline: 1.6510x; 1.1307x over previous
import functools

import jax
import jax.numpy as jnp
import numpy as np
from jax import lax
from jax.experimental import pallas as pl
from jax.experimental.pallas import tpu as pltpu
from jax.experimental.pallas import tpu_sc as plsc

N = 10000
E = 300000
H = 32
LAYERS = 4

NC = 2
NS = 16
NW = NC * NS
CHUNK = 128
NCHUNK = 74
PER_TILE = CHUNK * NCHUNK
EPAD = PER_TILE * NW
NROW = N // NS


@functools.lru_cache(maxsize=1)
def _mesh():
    return plsc.VectorSubcoreMesh(core_axis_name="c", subcore_axis_name="s")


def _sc_gather(table, idx):
    n, d = table.shape

    @functools.partial(
        pl.kernel,
        out_type=jax.ShapeDtypeStruct((EPAD, d), jnp.float32),
        mesh=_mesh(),
        compiler_params=pltpu.CompilerParams(use_tc_tiling_on_sc=False),
        scratch_types=[
            pltpu.VMEM((CHUNK,), jnp.int32),
            pltpu.VMEM((CHUNK, d), jnp.float32),
            pltpu.SemaphoreType.DMA,
        ],
    )
    def k(table_hbm, idx_hbm, out_hbm, idx_v, rows_v, sem):
        wid = lax.axis_index("s") * NC + lax.axis_index("c")

        def body(j, carry):
            base = wid * PER_TILE + j * CHUNK
            pltpu.sync_copy(idx_hbm.at[pl.ds(base, CHUNK)], idx_v)
            pltpu.async_copy(table_hbm.at[idx_v], rows_v, sem).wait()
            pltpu.sync_copy(rows_v, out_hbm.at[pl.ds(base, CHUNK)])
            return carry

        lax.fori_loop(0, NCHUNK, body, 0)

    return k(table, idx)


def _sc_scatter_add(rows, idx):
    d = rows.shape[1]
    zeros = jnp.zeros((N, d), jnp.float32)

    @functools.partial(
        pl.kernel,
        out_type=jax.ShapeDtypeStruct((NC, N, d), jnp.float32),
        mesh=_mesh(),
        compiler_params=pltpu.CompilerParams(use_tc_tiling_on_sc=False),
        scratch_types=[
            pltpu.VMEM((CHUNK,), jnp.int32),
            pltpu.VMEM((CHUNK, d), jnp.float32),
            pltpu.VMEM_SHARED((N, d), jnp.float32),
            pltpu.SemaphoreType.DMA,
        ],
    )
    def k(rows_hbm, idx_hbm, zero_hbm, out_hbm, idx_v, rows_v, acc_sh, sem):
        cid = lax.axis_index("c")
        sid = lax.axis_index("s")
        wid = sid * NC + cid
        pltpu.sync_copy(zero_hbm.at[pl.ds(sid * NROW, NROW)],
                        acc_sh.at[pl.ds(sid * NROW, NROW)])
        plsc.subcore_barrier()

        def body(j, carry):
            base = wid * PER_TILE + j * CHUNK
            pltpu.sync_copy(idx_hbm.at[pl.ds(base, CHUNK)], idx_v)
            pltpu.sync_copy(rows_hbm.at[pl.ds(base, CHUNK)], rows_v)
            pltpu.sync_copy(rows_v, acc_sh.at[idx_v], add=True)
            return carry

        lax.fori_loop(0, NCHUNK, body, 0)
        plsc.subcore_barrier()
        pltpu.sync_copy(acc_sh.at[pl.ds(sid * NROW, NROW)],
                        out_hbm.at[cid, pl.ds(sid * NROW, NROW)])

    return k(rows, idx, zeros)


def _segsum(rows, idx):
    parts = _sc_scatter_add(rows, idx)
    return parts[0] + parts[1]


def _normalize(v, axis=-1, eps=1e-8):
    return v / (jnp.linalg.norm(v, axis=axis, keepdims=True) + eps)


def _featurize(bb):
    X_ca = bb[:, 1, :]
    X = bb[:, :3, :].reshape(-1, 3)
    dX = X[1:] - X[:-1]
    U = _normalize(dX)
    u2 = U[:-2]
    u1 = U[1:-1]
    u0 = U[2:]
    n2 = _normalize(jnp.cross(u2, u1))
    n1 = _normalize(jnp.cross(u1, u0))
    cosD = jnp.clip(jnp.sum(n2 * n1, -1), -1 + 1e-7, 1 - 1e-7)
    D = jnp.sign(jnp.sum(u2 * n1, -1)) * jnp.arccos(cosD)
    D = jnp.pad(D, (1, 2)).reshape(-1, 3)
    dih = jnp.concatenate([jnp.cos(D), jnp.sin(D)], axis=-1)

    fwd = jnp.pad(_normalize(X_ca[1:] - X_ca[:-1]), ((0, 1), (0, 0)))
    bwd = jnp.pad(_normalize(X_ca[:-1] - X_ca[1:]), ((1, 0), (0, 0)))
    ori = jnp.stack([fwd, bwd], axis=-2)

    Nn = bb[:, 0, :]
    Ca = bb[:, 1, :]
    C = bb[:, 2, :]
    b = Ca - Nn
    c = C - Ca
    a = jnp.cross(b, c)
    vcb = (-0.58273431 * a + 0.56802827 * b - 0.54067466 * c + Ca) - X_ca

    bb_rel = bb - X_ca[:, None, :]
    l1 = jnp.nan_to_num(
        jnp.concatenate([bb_rel, ori, vcb[:, None, :]], axis=-2))

    emb = jnp.zeros((N, 4, 26), jnp.float32)
    emb = emb.at[:, 0, :6].set(dih)
    emb = emb.at[:, 1:4, :7].set(jnp.transpose(l1, (0, 2, 1)))
    return emb, X_ca


def _rbf(d):
    mu = jnp.linspace(0.0, 20.0, 16)
    sigma = 20.0 / 16
    return jnp.exp(-(((d[:, None] - mu) / sigma) ** 2))


def _pad_cols(x, d):
    return jnp.pad(x, ((0, 0), (0, d - x.shape[1])))


def kernel(bb, seq, latent, edge_index, params):
    src = edge_index[0].astype(jnp.int32)
    dst = edge_index[1].astype(jnp.int32)
    pad = EPAD - E
    src_p = jnp.pad(src, (0, pad))
    dst_p = jnp.pad(dst, (0, pad))
    emask = jnp.pad(jnp.ones((E,), jnp.float32), (0, pad))[:, None]

    emb, X_ca = _featurize(bb)

    xca16 = _pad_cols(X_ca, 16)
    xs = _sc_gather(xca16, src_p)[:, :3]
    xd = _sc_gather(xca16, dst_p)[:, :3]
    dist = jnp.linalg.norm(xd - xs, axis=-1)
    dpos = (src_p - dst_p).astype(jnp.float32)
    freq = jnp.exp(jnp.arange(0, 16, 2, dtype=jnp.float32) * (-np.log(10000.0) / 16))
    ang = dpos[:, None] * freq[None, :]
    e = jnp.concatenate([_rbf(dist), jnp.cos(ang), jnp.sin(ang)], axis=-1)

    emb28 = jnp.pad(emb, ((0, 0), (0, 0), (0, 2)))
    embflat = emb28.reshape(N, 112)
    gate = jax.nn.sigmoid(e @ params['Wg_bb'] + params['bg_bb'])
    rows = _sc_gather(embflat, src_p)
    w = rows * jnp.tile(_pad_cols(gate, 28), (1, 4)) * emask
    agg = _segsum(w, dst_p).reshape(N, 4, 28)[:, :, :26]
    bb_emb = agg @ params['Wo_bb'] + emb @ params['Ws_bb']

    x = jnp.concatenate([bb_emb, latent], axis=-1)

    for i in range(LAYERS):
        x_inv = x[:, 0, :]
        q = x_inv @ params['Wq'][i]
        kk = x_inv @ params['Wk'][i]
        qg = _sc_gather(q, dst_p)
        kg = _sc_gather(kk, src_p)
        score = jnp.sum(qg * kg, -1) / jnp.sqrt(16.0) + e @ params['We_att'][i]
        m = jnp.max(jnp.where(emask[:, 0] > 0, score, -1e30))
        ex = jnp.exp(score - m) * emask[:, 0]

        x_flat = x.reshape(N, 256)
        x_aug = jnp.concatenate(
            [x_flat, jnp.ones((N, 1), jnp.float32), jnp.zeros((N, 15), jnp.float32)],
            axis=-1)
        rows = _sc_gather(x_aug, src_p) * ex[:, None]
        agg = jnp.concatenate(
            [_segsum(rows[:, :144], dst_p), _segsum(rows[:, 144:], dst_p)], axis=-1)
        num = agg[:, :256].reshape(N, 4, 64)
        den = agg[:, 256:257]
        attn = (num / (den[:, :, None] + 1e-9)) @ params['Wv'][i]
        x = x + attn
        x = x + jax.nn.relu(x @ params['W1'][i]) @ params['W2'][i]

        x_inv = x[:, 0, :]
        ps = x_inv @ params['We1'][i][:64]
        pd = x_inv @ params['We1'][i][64:128]
        psg = _sc_gather(ps, src_p)
        pdg = _sc_gather(pd, dst_p)
        e = e + jax.nn.relu(psg + pdg + e @ params['We1'][i][128:] + params['be1'][i])

    x_flat = x.reshape(N, 256)
    gate = jax.nn.sigmoid(e @ params['Wg_p'] + params['bg_p'])
    rows = _sc_gather(x_flat, src_p) * jnp.tile(gate, (1, 4)) * emask
    agg = jnp.concatenate(
        [_segsum(rows[:, :128], dst_p), _segsum(rows[:, 128:], dst_p)], axis=-1)
    x2 = agg.reshape(N, 4, 64) @ params['Wo_p'] + x @ params['Ws_p']

    x2_flat = x2.reshape(N, 128)
    gate = jax.nn.sigmoid(e @ params['Wg_a'] + params['bg_a'])
    rows = _sc_gather(x2_flat, src_p) * jnp.tile(gate, (1, 4)) * emask
    agg = _segsum(rows, dst_p).reshape(N, 4, 32)
    a = agg @ params['Wo_a'] + x2 @ params['Ws_a']

    inv = a[:, 0, :]
    mu = jnp.mean(inv, -1, keepdims=True)
    var = jnp.var(inv, -1, keepdims=True)
    ln = (inv - mu) / jnp.sqrt(var + 1e-5) * params['ln_g'] + params['ln_b']
    seq_logits = jax.nn.log_softmax(ln @ params['W_seq'] + params['b_seq'], axis=-1)
    decoded_latent = jnp.transpose(a[:, 1:4, :], (0, 2, 1))
    return decoded_latent, seq_logits


# speedup vs baseline: 1.7235x; 1.0439x over previous
import functools

import jax
import jax.numpy as jnp
import numpy as np
from jax import lax
from jax.experimental import pallas as pl
from jax.experimental.pallas import tpu as pltpu
from jax.experimental.pallas import tpu_sc as plsc

N = 10000
E = 300000
H = 32
LAYERS = 4

NC = 2
NS = 16
NW = NC * NS
CHUNK = 128
NCHUNK = 74
PER_TILE = CHUNK * NCHUNK
EPAD = PER_TILE * NW
NROW = N // NS


@functools.lru_cache(maxsize=1)
def _mesh():
    return plsc.VectorSubcoreMesh(core_axis_name="c", subcore_axis_name="s")


def _sc_gather(table, idx):
    n, d = table.shape

    @functools.partial(
        pl.kernel,
        out_type=jax.ShapeDtypeStruct((EPAD, d), jnp.float32),
        mesh=_mesh(),
        compiler_params=pltpu.CompilerParams(use_tc_tiling_on_sc=False),
        scratch_types=[
            pltpu.VMEM((PER_TILE,), jnp.int32),
            pltpu.VMEM((CHUNK, d), jnp.float32),
            pltpu.SemaphoreType.DMA,
        ],
    )
    def k(table_hbm, idx_hbm, out_hbm, idx_v, rows_v, sem):
        wid = lax.axis_index("s") * NC + lax.axis_index("c")
        pltpu.sync_copy(idx_hbm.at[pl.ds(wid * PER_TILE, PER_TILE)], idx_v)

        def body(j, carry):
            base = wid * PER_TILE + j * CHUNK
            pltpu.async_copy(table_hbm.at[idx_v.at[pl.ds(j * CHUNK, CHUNK)]],
                             rows_v, sem).wait()
            pltpu.sync_copy(rows_v, out_hbm.at[pl.ds(base, CHUNK)])
            return carry

        lax.fori_loop(0, NCHUNK, body, 0)

    return k(table, idx)


def _sc_scatter_add(rows, idx):
    d = rows.shape[1]
    zeros = jnp.zeros((N, d), jnp.float32)

    @functools.partial(
        pl.kernel,
        out_type=jax.ShapeDtypeStruct((NC, N, d), jnp.float32),
        mesh=_mesh(),
        compiler_params=pltpu.CompilerParams(use_tc_tiling_on_sc=False),
        scratch_types=[
            pltpu.VMEM((PER_TILE,), jnp.int32),
            pltpu.VMEM((CHUNK,), jnp.int32),
            pltpu.VMEM((CHUNK, d), jnp.float32),
            pltpu.VMEM_SHARED((N, d), jnp.float32),
            pltpu.SemaphoreType.DMA,
        ],
    )
    def k(rows_hbm, idx_hbm, zero_hbm, out_hbm, idx_s, idx_v, rows_v,
          acc_sh, sem):
        cid = lax.axis_index("c")
        sid = lax.axis_index("s")
        wid = sid * NC + cid
        pltpu.sync_copy(idx_hbm.at[pl.ds(wid * PER_TILE, PER_TILE)], idx_s)
        pltpu.sync_copy(zero_hbm.at[pl.ds(sid * NROW, NROW)],
                        acc_sh.at[pl.ds(sid * NROW, NROW)])
        plsc.subcore_barrier()

        def body(j, carry):
            base = wid * PER_TILE + j * CHUNK
            ld = pltpu.async_copy(rows_hbm.at[pl.ds(base, CHUNK)], rows_v,
                                  sem)
            for i in range(CHUNK // 16):
                idx_v[pl.ds(i * 16, 16)] = idx_s[pl.ds(j * CHUNK + i * 16, 16)]
            ld.wait()
            pltpu.sync_copy(rows_v, acc_sh.at[idx_v], add=True)
            return carry

        lax.fori_loop(0, NCHUNK, body, 0)
        plsc.subcore_barrier()
        pltpu.sync_copy(acc_sh.at[pl.ds(sid * NROW, NROW)],
                        out_hbm.at[cid, pl.ds(sid * NROW, NROW)])

    return k(rows, idx, zeros)


def _segsum(rows, idx):
    parts = _sc_scatter_add(rows, idx)
    return parts[0] + parts[1]


def _normalize(v, axis=-1, eps=1e-8):
    return v / (jnp.linalg.norm(v, axis=axis, keepdims=True) + eps)


def _featurize(bb):
    X_ca = bb[:, 1, :]
    X = bb[:, :3, :].reshape(-1, 3)
    dX = X[1:] - X[:-1]
    U = _normalize(dX)
    u2 = U[:-2]
    u1 = U[1:-1]
    u0 = U[2:]
    n2 = _normalize(jnp.cross(u2, u1))
    n1 = _normalize(jnp.cross(u1, u0))
    cosD = jnp.clip(jnp.sum(n2 * n1, -1), -1 + 1e-7, 1 - 1e-7)
    D = jnp.sign(jnp.sum(u2 * n1, -1)) * jnp.arccos(cosD)
    D = jnp.pad(D, (1, 2)).reshape(-1, 3)
    dih = jnp.concatenate([jnp.cos(D), jnp.sin(D)], axis=-1)

    fwd = jnp.pad(_normalize(X_ca[1:] - X_ca[:-1]), ((0, 1), (0, 0)))
    bwd = jnp.pad(_normalize(X_ca[:-1] - X_ca[1:]), ((1, 0), (0, 0)))
    ori = jnp.stack([fwd, bwd], axis=-2)

    Nn = bb[:, 0, :]
    Ca = bb[:, 1, :]
    C = bb[:, 2, :]
    b = Ca - Nn
    c = C - Ca
    a = jnp.cross(b, c)
    vcb = (-0.58273431 * a + 0.56802827 * b - 0.54067466 * c + Ca) - X_ca

    bb_rel = bb - X_ca[:, None, :]
    l1 = jnp.nan_to_num(
        jnp.concatenate([bb_rel, ori, vcb[:, None, :]], axis=-2))

    emb = jnp.zeros((N, 4, 26), jnp.float32)
    emb = emb.at[:, 0, :6].set(dih)
    emb = emb.at[:, 1:4, :7].set(jnp.transpose(l1, (0, 2, 1)))
    return emb, X_ca


def _rbf(d):
    mu = jnp.linspace(0.0, 20.0, 16)
    sigma = 20.0 / 16
    return jnp.exp(-(((d[:, None] - mu) / sigma) ** 2))


def _pad_cols(x, d):
    return jnp.pad(x, ((0, 0), (0, d - x.shape[1])))


def kernel(bb, seq, latent, edge_index, params):
    src = edge_index[0].astype(jnp.int32)
    dst = edge_index[1].astype(jnp.int32)
    pad = EPAD - E
    src_p = jnp.pad(src, (0, pad))
    dst_p = jnp.pad(dst, (0, pad))
    emask = jnp.pad(jnp.ones((E,), jnp.float32), (0, pad))[:, None]

    emb, X_ca = _featurize(bb)

    xca16 = _pad_cols(X_ca, 16)
    xs = _sc_gather(xca16, src_p)[:, :3]
    xd = _sc_gather(xca16, dst_p)[:, :3]
    dist = jnp.linalg.norm(xd - xs, axis=-1)
    dpos = (src_p - dst_p).astype(jnp.float32)
    freq = jnp.exp(jnp.arange(0, 16, 2, dtype=jnp.float32) * (-np.log(10000.0) / 16))
    ang = dpos[:, None] * freq[None, :]
    e = jnp.concatenate([_rbf(dist), jnp.cos(ang), jnp.sin(ang)], axis=-1)

    emb28 = jnp.pad(emb, ((0, 0), (0, 0), (0, 2)))
    embflat = emb28.reshape(N, 112)
    gate = jax.nn.sigmoid(e @ params['Wg_bb'] + params['bg_bb'])
    rows = _sc_gather(embflat, src_p)
    w = rows * jnp.tile(_pad_cols(gate, 28), (1, 4)) * emask
    agg = _segsum(w, dst_p).reshape(N, 4, 28)[:, :, :26]
    bb_emb = agg @ params['Wo_bb'] + emb @ params['Ws_bb']

    x = jnp.concatenate([bb_emb, latent], axis=-1)

    for i in range(LAYERS):
        x_inv = x[:, 0, :]
        q = x_inv @ params['Wq'][i]
        kk = x_inv @ params['Wk'][i]
        qg = _sc_gather(q, dst_p)
        kg = _sc_gather(kk, src_p)
        score = jnp.sum(qg * kg, -1) / jnp.sqrt(16.0) + e @ params['We_att'][i]
        m = jnp.max(jnp.where(emask[:, 0] > 0, score, -1e30))
        ex = jnp.exp(score - m) * emask[:, 0]

        x_flat = x.reshape(N, 256)
        x_aug = jnp.concatenate(
            [x_flat, jnp.ones((N, 1), jnp.float32), jnp.zeros((N, 15), jnp.float32)],
            axis=-1)
        rows = _sc_gather(x_aug, src_p) * ex[:, None]
        agg = jnp.concatenate(
            [_segsum(rows[:, :144], dst_p), _segsum(rows[:, 144:], dst_p)], axis=-1)
        num = agg[:, :256].reshape(N, 4, 64)
        den = agg[:, 256:257]
        attn = (num / (den[:, :, None] + 1e-9)) @ params['Wv'][i]
        x = x + attn
        x = x + jax.nn.relu(x @ params['W1'][i]) @ params['W2'][i]

        x_inv = x[:, 0, :]
        ps = x_inv @ params['We1'][i][:64]
        pd = x_inv @ params['We1'][i][64:128]
        psg = _sc_gather(ps, src_p)
        pdg = _sc_gather(pd, dst_p)
        e = e + jax.nn.relu(psg + pdg + e @ params['We1'][i][128:] + params['be1'][i])

    x_flat = x.reshape(N, 256)
    gate = jax.nn.sigmoid(e @ params['Wg_p'] + params['bg_p'])
    rows = _sc_gather(x_flat, src_p) * jnp.tile(gate, (1, 4)) * emask
    agg = jnp.concatenate(
        [_segsum(rows[:, :128], dst_p), _segsum(rows[:, 128:], dst_p)], axis=-1)
    x2 = agg.reshape(N, 4, 64) @ params['Wo_p'] + x @ params['Ws_p']

    x2_flat = x2.reshape(N, 128)
    gate = jax.nn.sigmoid(e @ params['Wg_a'] + params['bg_a'])
    rows = _sc_gather(x2_flat, src_p) * jnp.tile(gate, (1, 4)) * emask
    agg = _segsum(rows, dst_p).reshape(N, 4, 32)
    a = agg @ params['Wo_a'] + x2 @ params['Ws_a']

    inv = a[:, 0, :]
    mu = jnp.mean(inv, -1, keepdims=True)
    var = jnp.var(inv, -1, keepdims=True)
    ln = (inv - mu) / jnp.sqrt(var + 1e-5) * params['ln_g'] + params['ln_b']
    seq_logits = jax.nn.log_softmax(ln @ params['W_seq'] + params['b_seq'], axis=-1)
    decoded_latent = jnp.transpose(a[:, 1:4, :], (0, 2, 1))
    return decoded_latent, seq_logits


# speedup vs baseline: 1.7673x; 1.0255x over previous
import functools

import jax
import jax.numpy as jnp
import numpy as np
from jax import lax
from jax.experimental import pallas as pl
from jax.experimental.pallas import tpu as pltpu
from jax.experimental.pallas import tpu_sc as plsc

N = 10000
E = 300000
H = 32
LAYERS = 4

NC = 2
NS = 16
NW = NC * NS
CHUNK = 128
NCHUNK = 74
PER_TILE = CHUNK * NCHUNK
EPAD = PER_TILE * NW
NROW = N // NS


@functools.lru_cache(maxsize=1)
def _mesh():
    return plsc.VectorSubcoreMesh(core_axis_name="c", subcore_axis_name="s")


def _sc_gather(table, idx):
    n, d = table.shape

    @functools.partial(
        pl.kernel,
        out_type=jax.ShapeDtypeStruct((EPAD, d), jnp.float32),
        mesh=_mesh(),
        compiler_params=pltpu.CompilerParams(use_tc_tiling_on_sc=False),
        scratch_types=[
            pltpu.VMEM((PER_TILE,), jnp.int32),
            pltpu.VMEM((CHUNK, d), jnp.float32),
            pltpu.VMEM((CHUNK, d), jnp.float32),
            pltpu.SemaphoreType.DMA,
            pltpu.SemaphoreType.DMA,
            pltpu.SemaphoreType.DMA,
            pltpu.SemaphoreType.DMA,
        ],
    )
    def k(table_hbm, idx_hbm, out_hbm, idx_v, rows_a, rows_b,
          sga, sgb, ssa, ssb):
        wid = lax.axis_index("s") * NC + lax.axis_index("c")
        pltpu.sync_copy(idx_hbm.at[pl.ds(wid * PER_TILE, PER_TILE)], idx_v)

        def body(j, carry):
            base = wid * PER_TILE + 2 * j * CHUNK
            ga = pltpu.async_copy(
                table_hbm.at[idx_v.at[pl.ds(2 * j * CHUNK, CHUNK)]],
                rows_a, sga)
            gb = pltpu.async_copy(
                table_hbm.at[idx_v.at[pl.ds((2 * j + 1) * CHUNK, CHUNK)]],
                rows_b, sgb)
            ga.wait()
            sa = pltpu.async_copy(rows_a, out_hbm.at[pl.ds(base, CHUNK)], ssa)
            gb.wait()
            sb = pltpu.async_copy(rows_b, out_hbm.at[pl.ds(base + CHUNK, CHUNK)],
                                  ssb)
            sa.wait()
            sb.wait()
            return carry

        lax.fori_loop(0, NCHUNK // 2, body, 0)

    return k(table, idx)


def _sc_scatter_add(rows, idx):
    d = rows.shape[1]
    zeros = jnp.zeros((N, d), jnp.float32)

    @functools.partial(
        pl.kernel,
        out_type=jax.ShapeDtypeStruct((NC, N, d), jnp.float32),
        mesh=_mesh(),
        compiler_params=pltpu.CompilerParams(use_tc_tiling_on_sc=False),
        scratch_types=[
            pltpu.VMEM((PER_TILE,), jnp.int32),
            pltpu.VMEM((CHUNK,), jnp.int32),
            pltpu.VMEM((CHUNK, d), jnp.float32),
            pltpu.VMEM_SHARED((N, d), jnp.float32),
            pltpu.SemaphoreType.DMA,
        ],
    )
    def k(rows_hbm, idx_hbm, zero_hbm, out_hbm, idx_s, idx_v, rows_v,
          acc_sh, sem):
        cid = lax.axis_index("c")
        sid = lax.axis_index("s")
        wid = sid * NC + cid
        pltpu.sync_copy(idx_hbm.at[pl.ds(wid * PER_TILE, PER_TILE)], idx_s)
        pltpu.sync_copy(zero_hbm.at[pl.ds(sid * NROW, NROW)],
                        acc_sh.at[pl.ds(sid * NROW, NROW)])
        plsc.subcore_barrier()

        def body(j, carry):
            base = wid * PER_TILE + j * CHUNK
            ld = pltpu.async_copy(rows_hbm.at[pl.ds(base, CHUNK)], rows_v,
                                  sem)
            for i in range(CHUNK // 16):
                idx_v[pl.ds(i * 16, 16)] = idx_s[pl.ds(j * CHUNK + i * 16, 16)]
            ld.wait()
            pltpu.sync_copy(rows_v, acc_sh.at[idx_v], add=True)
            return carry

        lax.fori_loop(0, NCHUNK, body, 0)
        plsc.subcore_barrier()
        pltpu.sync_copy(acc_sh.at[pl.ds(sid * NROW, NROW)],
                        out_hbm.at[cid, pl.ds(sid * NROW, NROW)])

    return k(rows, idx, zeros)


def _segsum(rows, idx):
    parts = _sc_scatter_add(rows, idx)
    return parts[0] + parts[1]


def _normalize(v, axis=-1, eps=1e-8):
    return v / (jnp.linalg.norm(v, axis=axis, keepdims=True) + eps)


def _featurize(bb):
    X_ca = bb[:, 1, :]
    X = bb[:, :3, :].reshape(-1, 3)
    dX = X[1:] - X[:-1]
    U = _normalize(dX)
    u2 = U[:-2]
    u1 = U[1:-1]
    u0 = U[2:]
    n2 = _normalize(jnp.cross(u2, u1))
    n1 = _normalize(jnp.cross(u1, u0))
    cosD = jnp.clip(jnp.sum(n2 * n1, -1), -1 + 1e-7, 1 - 1e-7)
    D = jnp.sign(jnp.sum(u2 * n1, -1)) * jnp.arccos(cosD)
    D = jnp.pad(D, (1, 2)).reshape(-1, 3)
    dih = jnp.concatenate([jnp.cos(D), jnp.sin(D)], axis=-1)

    fwd = jnp.pad(_normalize(X_ca[1:] - X_ca[:-1]), ((0, 1), (0, 0)))
    bwd = jnp.pad(_normalize(X_ca[:-1] - X_ca[1:]), ((1, 0), (0, 0)))
    ori = jnp.stack([fwd, bwd], axis=-2)

    Nn = bb[:, 0, :]
    Ca = bb[:, 1, :]
    C = bb[:, 2, :]
    b = Ca - Nn
    c = C - Ca
    a = jnp.cross(b, c)
    vcb = (-0.58273431 * a + 0.56802827 * b - 0.54067466 * c + Ca) - X_ca

    bb_rel = bb - X_ca[:, None, :]
    l1 = jnp.nan_to_num(
        jnp.concatenate([bb_rel, ori, vcb[:, None, :]], axis=-2))

    emb = jnp.zeros((N, 4, 26), jnp.float32)
    emb = emb.at[:, 0, :6].set(dih)
    emb = emb.at[:, 1:4, :7].set(jnp.transpose(l1, (0, 2, 1)))
    return emb, X_ca


def _rbf(d):
    mu = jnp.linspace(0.0, 20.0, 16)
    sigma = 20.0 / 16
    return jnp.exp(-(((d[:, None] - mu) / sigma) ** 2))


def _pad_cols(x, d):
    return jnp.pad(x, ((0, 0), (0, d - x.shape[1])))


def kernel(bb, seq, latent, edge_index, params):
    src = edge_index[0].astype(jnp.int32)
    dst = edge_index[1].astype(jnp.int32)
    pad = EPAD - E
    src_p = jnp.pad(src, (0, pad))
    dst_p = jnp.pad(dst, (0, pad))
    emask = jnp.pad(jnp.ones((E,), jnp.float32), (0, pad))[:, None]

    emb, X_ca = _featurize(bb)

    xca16 = _pad_cols(X_ca, 16)
    xs = _sc_gather(xca16, src_p)[:, :3]
    xd = _sc_gather(xca16, dst_p)[:, :3]
    dist = jnp.linalg.norm(xd - xs, axis=-1)
    dpos = (src_p - dst_p).astype(jnp.float32)
    freq = jnp.exp(jnp.arange(0, 16, 2, dtype=jnp.float32) * (-np.log(10000.0) / 16))
    ang = dpos[:, None] * freq[None, :]
    e = jnp.concatenate([_rbf(dist), jnp.cos(ang), jnp.sin(ang)], axis=-1)

    emb28 = jnp.pad(emb, ((0, 0), (0, 0), (0, 2)))
    embflat = emb28.reshape(N, 112)
    gate = jax.nn.sigmoid(e @ params['Wg_bb'] + params['bg_bb'])
    rows = _sc_gather(embflat, src_p)
    w = rows * jnp.tile(_pad_cols(gate, 28), (1, 4)) * emask
    agg = _segsum(w, dst_p).reshape(N, 4, 28)[:, :, :26]
    bb_emb = agg @ params['Wo_bb'] + emb @ params['Ws_bb']

    x = jnp.concatenate([bb_emb, latent], axis=-1)

    for i in range(LAYERS):
        x_inv = x[:, 0, :]
        q = x_inv @ params['Wq'][i]
        kk = x_inv @ params['Wk'][i]
        qg = _sc_gather(q, dst_p)
        kg = _sc_gather(kk, src_p)
        score = jnp.sum(qg * kg, -1) / jnp.sqrt(16.0) + e @ params['We_att'][i]
        m = jnp.max(jnp.where(emask[:, 0] > 0, score, -1e30))
        ex = jnp.exp(score - m) * emask[:, 0]

        x_flat = x.reshape(N, 256)
        x_aug = jnp.concatenate(
            [x_flat, jnp.ones((N, 1), jnp.float32), jnp.zeros((N, 15), jnp.float32)],
            axis=-1)
        rows = _sc_gather(x_aug, src_p) * ex[:, None]
        agg = jnp.concatenate(
            [_segsum(rows[:, :144], dst_p), _segsum(rows[:, 144:], dst_p)], axis=-1)
        num = agg[:, :256].reshape(N, 4, 64)
        den = agg[:, 256:257]
        attn = (num / (den[:, :, None] + 1e-9)) @ params['Wv'][i]
        x = x + attn
        x = x + jax.nn.relu(x @ params['W1'][i]) @ params['W2'][i]

        x_inv = x[:, 0, :]
        ps = x_inv @ params['We1'][i][:64]
        pd = x_inv @ params['We1'][i][64:128]
        psg = _sc_gather(ps, src_p)
        pdg = _sc_gather(pd, dst_p)
        e = e + jax.nn.relu(psg + pdg + e @ params['We1'][i][128:] + params['be1'][i])

    x_flat = x.reshape(N, 256)
    gate = jax.nn.sigmoid(e @ params['Wg_p'] + params['bg_p'])
    rows = _sc_gather(x_flat, src_p) * jnp.tile(gate, (1, 4)) * emask
    agg = jnp.concatenate(
        [_segsum(rows[:, :128], dst_p), _segsum(rows[:, 128:], dst_p)], axis=-1)
    x2 = agg.reshape(N, 4, 64) @ params['Wo_p'] + x @ params['Ws_p']

    x2_flat = x2.reshape(N, 128)
    gate = jax.nn.sigmoid(e @ params['Wg_a'] + params['bg_a'])
    rows = _sc_gather(x2_flat, src_p) * jnp.tile(gate, (1, 4)) * emask
    agg = _segsum(rows, dst_p).reshape(N, 4, 32)
    a = agg @ params['Wo_a'] + x2 @ params['Ws_a']

    inv = a[:, 0, :]
    mu = jnp.mean(inv, -1, keepdims=True)
    var = jnp.var(inv, -1, keepdims=True)
    ln = (inv - mu) / jnp.sqrt(var + 1e-5) * params['ln_g'] + params['ln_b']
    seq_logits = jax.nn.log_softmax(ln @ params['W_seq'] + params['b_seq'], axis=-1)
    decoded_latent = jnp.transpose(a[:, 1:4, :], (0, 2, 1))
    return decoded_latent, seq_logits
